# baseline (speedup 1.0000x reference)
import functools

import jax
import jax.numpy as jnp
from jax import lax
from jax.experimental import pallas as pl
from jax.experimental.pallas import tpu as pltpu
from jax.experimental.pallas import tpu_sc as plsc

F32 = jnp.float32

N = 10000
E = 160000
F = 128
H = 4

NC = 2
NS = 16
NW = NC * NS
EPW = E // NW
CB = 40
NCH = EPW // CB
QW = H * F
ZCH = 64
NZC = -(-N // ZCH)
CB2 = 32
NCH2 = 156
XCH2 = 8


def _prep_kernel(wproj_ref, wout_ref, rs_ref, ts_ref, a2_ref, vt_ref):
    wout = wout_ref[...]
    cols_r = []
    cols_t = []
    for h in range(H):
        wh = wproj_ref[h]
        a2_ref[:, h * F:(h + 1) * F] = jnp.dot(
            wh, wout, preferred_element_type=F32)
        cols_r.append(jnp.dot(wh, rs_ref[h][:, None],
                              preferred_element_type=F32))
        cols_t.append(jnp.dot(wh, ts_ref[h][:, None],
                              preferred_element_type=F32))
    pad = jnp.zeros((F, F - 8), F32)
    vt_ref[...] = jnp.concatenate(cols_r + cols_t + [pad], axis=1)


def _proj_kernel(x_ref, a2_ref, vt_ref, q_ref, s_ref):
    xb = x_ref[...]
    q_ref[...] = jnp.dot(xb, a2_ref[...], preferred_element_type=F32)
    s_ref[...] = jnp.dot(xb, vt_ref[...], preferred_element_type=F32)


def _pass1_body(s_hbm, snd_hbm, rcv_hbm, len_hbm, ew_hbm, rsum_hbm,
                sidx, ridx, ssb, ewb, ewout, lenb, zb, accum, sem, sem2):
    cid = lax.axis_index("c")
    sid = lax.axis_index("s")
    wid = sid * NC + cid

    zrow = jnp.zeros((16,), F32)

    def zero_wide(i, _):
        for j in range(8):
            zb[i, pl.ds(j * 16, 16)] = zrow
        return _

    def zero_ewb(i, _):
        for j in range(8):
            ewb[i, pl.ds(j * 16, 16)] = zrow
        return _

    lax.fori_loop(0, CB, zero_ewb, None)
    lax.fori_loop(0, ZCH, zero_wide, None)
    for c in range(NZC):
        sz = min(ZCH, N - c * ZCH)

        @pl.when(sid == c % NS)
        def _():
            pltpu.sync_copy(zb.at[pl.ds(0, sz)], accum.at[pl.ds(c * ZCH, sz)])
    plsc.subcore_barrier()

    base0 = wid * EPW
    mask = jnp.where(lax.broadcasted_iota(jnp.int32, (16,), 0) < 4,
                     jnp.float32(1.0), jnp.float32(0.0))

    def chunk(c, _):
        base = base0 + c * CB
        d_s = pltpu.async_copy(snd_hbm.at[pl.ds(base, CB)], sidx, sem2)
        d_r = pltpu.async_copy(rcv_hbm.at[pl.ds(base, CB)], ridx, sem)
        d_l = pltpu.async_copy(len_hbm.at[pl.ds(base, CB)],
                               lenb.at[pl.ds(0, CB)], sem)
        d_s.wait()
        d_g = pltpu.async_copy(s_hbm.at[sidx], ssb, sem)
        d_r.wait()
        d_l.wait()
        d_g.wait()

        def edge(i, _):
            lv = lenb[pl.ds(i, 16)][0]
            row = jnp.exp(ssb[i, pl.ds(0, 16)] - lv * mask)
            ewb[i, pl.ds(0, 16)] = row
            ewout[i] = row
            return _

        lax.fori_loop(0, CB, edge, None, unroll=4)
        pltpu.sync_copy(ewout, ew_hbm.at[pl.ds(base, CB)])
        pltpu.sync_copy(ewb, accum.at[ridx], add=True)
        return _

    lax.fori_loop(0, NCH, chunk, None)
    plsc.subcore_barrier()
    for c in range(NZC):
        sz = min(ZCH, N - c * ZCH)

        @pl.when(sid == c % NS)
        def _():
            pltpu.sync_copy(accum.at[pl.ds(c * ZCH, sz)],
                            rsum_hbm.at[pl.ds(cid * N + c * ZCH, sz)])


def _combine_kernel(p0_ref, p1_ref, inv_ref):
    inv_ref[...] = 1.0 / (p0_ref[...] + p1_ref[...])


def _pass2_body(q_hbm, inv_hbm, ew_hbm, snd_hbm, rcv_hbm, out_hbm,
                sidx0, sidx1, ridx0, ridx1, qrows0, qrows1, ewb0, ewb1,
                invb, outb, accum, gsem0, gsem1, isem0, isem1):
    cid = lax.axis_index("c")
    sid = lax.axis_index("s")
    wid = sid * NC + cid
    nch = jnp.where(wid == NW - 1, NCH2 + XCH2, NCH2)

    zrow = jnp.zeros((16,), F32)

    def zero_row(i, _):
        for j in range(8):
            outb[i, pl.ds(j * 16, 16)] = zrow
        return _

    lax.fori_loop(0, CB2, zero_row, None)

    def zero_chunk(c, _):
        @pl.when(c % NS == sid)
        def _():
            pltpu.sync_copy(outb, accum.at[pl.ds(c * CB2, CB2)])
        return _

    lax.fori_loop(0, N // CB2, zero_chunk, None)
    if N % CB2:
        @pl.when(sid == (N // CB2) % NS)
        def _():
            pltpu.sync_copy(outb.at[pl.ds(0, N % CB2)],
                            accum.at[pl.ds(N - N % CB2, N % CB2)])
    plsc.subcore_barrier()

    base0 = wid * (NCH2 * CB2)
    slots = ((sidx0, ridx0, qrows0, ewb0, gsem0, isem0),
             (sidx1, ridx1, qrows1, ewb1, gsem1, isem1))

    def issue(c, slot):
        sidx, ridx, qrows, ewb, gsem, isem = slots[slot]
        base = base0 + c * CB2
        d_s = pltpu.async_copy(snd_hbm.at[pl.ds(base, CB2)], sidx, isem)
        d_r = pltpu.async_copy(rcv_hbm.at[pl.ds(base, CB2)], ridx, isem)
        pltpu.async_copy(ew_hbm.at[pl.ds(base, CB2)], ewb, gsem)
        d_s.wait()
        d_r.wait()
        pltpu.async_copy(q_hbm.at[sidx], qrows, gsem)

    def process(c, slot):
        sidx, ridx, qrows, ewb, gsem, isem = slots[slot]
        d_i = pltpu.async_copy(inv_hbm.at[ridx], invb, gsem)
        pltpu.make_async_copy(ew_hbm.at[pl.ds(0, CB2)], ewb, gsem).wait()
        pltpu.make_async_copy(q_hbm.at[pl.ds(0, CB2)], qrows, gsem).wait()
        d_i.wait()

        def edge(i, _):
            wv = ewb[i] * invb[i, pl.ds(0, 16)]
            w0 = wv[0] + wv[4]
            w1 = wv[1] + wv[5]
            w2 = wv[2] + wv[6]
            w3 = wv[3] + wv[7]
            for j in range(8):
                o = j * 16
                acc = (w0 * qrows[i, pl.ds(o, 16)]
                       + w1 * qrows[i, pl.ds(F + o, 16)]
                       + w2 * qrows[i, pl.ds(2 * F + o, 16)]
                       + w3 * qrows[i, pl.ds(3 * F + o, 16)])
                outb[i, pl.ds(o, 16)] = acc
            return _

        lax.fori_loop(0, CB2, edge, None, unroll=4)
        pltpu.sync_copy(outb, accum.at[ridx], add=True)

        @pl.when(c + 2 < nch)
        def _():
            issue(c + 2, slot)

    @pl.when(nch > 0)
    def _():
        issue(0, 0)

    @pl.when(nch > 1)
    def _():
        issue(1, 1)

    def pair(g, _):
        c = 2 * g

        @pl.when(c < nch)
        def _():
            process(c, 0)

        @pl.when(c + 1 < nch)
        def _():
            process(c + 1, 1)
        return _

    lax.fori_loop(0, (NCH2 + XCH2 + 1) // 2, pair, None)
    plsc.subcore_barrier()

    def dump_chunk(c, _):
        @pl.when(c % NS == sid)
        def _():
            pltpu.sync_copy(accum.at[pl.ds(c * CB2, CB2)],
                            out_hbm.at[pl.ds(cid * N + c * CB2, CB2)])
        return _

    lax.fori_loop(0, N // CB2, dump_chunk, None)
    if N % CB2:
        @pl.when(sid == (N // CB2) % NS)
        def _():
            pltpu.sync_copy(accum.at[pl.ds(N - N % CB2, N % CB2)],
                            out_hbm.at[pl.ds(cid * N + N - N % CB2,
                                             N % CB2)])


def _final_kernel(x_ref, q_ref, o0_ref, o1_ref, inv_ref, out_ref):
    ind = (inv_ref[...][:, :1] < jnp.inf).astype(F32)
    q = q_ref[...]
    sq = (q[:, 0 * F:1 * F] + q[:, 1 * F:2 * F]
          + q[:, 2 * F:3 * F] + q[:, 3 * F:4 * F])
    acc = o0_ref[...] + o1_ref[...] - 2.0 * ind * sq
    out_ref[...] = x_ref[...] + acc * (1.0 / H)


BN = 400


@jax.jit
def kernel(x, edge_index, edge_vec, edge_len, W_proj, W_out,
           radial_score, tangential_score, radial_distance_scale):
    del edge_vec
    snd = edge_index[0]
    rcv = edge_index[1]
    len2 = edge_len * radial_distance_scale

    a2, vt = pl.pallas_call(
        _prep_kernel,
        out_shape=(jax.ShapeDtypeStruct((F, QW), F32),
                   jax.ShapeDtypeStruct((F, F), F32)),
    )(W_proj, W_out, radial_score, tangential_score)

    nb = N // BN
    q, s = pl.pallas_call(
        _proj_kernel,
        grid=(nb,),
        in_specs=[pl.BlockSpec((BN, F), lambda i: (i, 0)),
                  pl.BlockSpec((F, QW), lambda i: (0, 0)),
                  pl.BlockSpec((F, F), lambda i: (0, 0))],
        out_specs=(pl.BlockSpec((BN, QW), lambda i: (i, 0)),
                   pl.BlockSpec((BN, F), lambda i: (i, 0))),
        out_shape=(jax.ShapeDtypeStruct((N, QW), F32),
                   jax.ShapeDtypeStruct((N, F), F32)),
    )(x, a2, vt)

    mesh = plsc.VectorSubcoreMesh(core_axis_name="c", subcore_axis_name="s")

    pass1 = functools.partial(
        pl.kernel,
        out_type=(jax.ShapeDtypeStruct((E, 16), F32),
                  jax.ShapeDtypeStruct((NC * N, F), F32)),
        mesh=mesh,
        scratch_types=[
            pltpu.VMEM((CB,), jnp.int32),
            pltpu.VMEM((CB,), jnp.int32),
            pltpu.VMEM((CB, F), F32),
            pltpu.VMEM((CB, F), F32),
            pltpu.VMEM((CB, 16), F32),
            pltpu.VMEM((CB + 16,), F32),
            pltpu.VMEM((ZCH, F), F32),
            pltpu.VMEM_SHARED((N, F), F32),
            pltpu.SemaphoreType.DMA,
            pltpu.SemaphoreType.DMA,
        ],
    )(_pass1_body)
    ew, rsum_parts = pass1(s, snd, rcv, len2)

    nb = N // BN
    inv = pl.pallas_call(
        _combine_kernel,
        grid=(nb,),
        in_specs=[pl.BlockSpec((BN, F), lambda i: (i, 0)),
                  pl.BlockSpec((BN, F), lambda i: (i + nb, 0))],
        out_specs=pl.BlockSpec((BN, F), lambda i: (i, 0)),
        out_shape=jax.ShapeDtypeStruct((N, F), F32),
    )(rsum_parts, rsum_parts)

    pass2 = functools.partial(
        pl.kernel,
        out_type=jax.ShapeDtypeStruct((NC * N, F), F32),
        mesh=mesh,
        scratch_types=[
            pltpu.VMEM((CB2,), jnp.int32),
            pltpu.VMEM((CB2,), jnp.int32),
            pltpu.VMEM((CB2,), jnp.int32),
            pltpu.VMEM((CB2,), jnp.int32),
            pltpu.VMEM((CB2, QW), F32),
            pltpu.VMEM((CB2, QW), F32),
            pltpu.VMEM((CB2, 16), F32),
            pltpu.VMEM((CB2, 16), F32),
            pltpu.VMEM((CB2, F), F32),
            pltpu.VMEM((CB2, F), F32),
            pltpu.VMEM_SHARED((N, F), F32),
            pltpu.SemaphoreType.DMA,
            pltpu.SemaphoreType.DMA,
            pltpu.SemaphoreType.DMA,
            pltpu.SemaphoreType.DMA,
        ],
    )(_pass2_body)
    out_parts = pass2(q, inv, ew, snd, rcv)

    out = pl.pallas_call(
        _final_kernel,
        grid=(nb,),
        in_specs=[pl.BlockSpec((BN, F), lambda i: (i, 0)),
                  pl.BlockSpec((BN, QW), lambda i: (i, 0)),
                  pl.BlockSpec((BN, F), lambda i: (i, 0)),
                  pl.BlockSpec((BN, F), lambda i: (i + nb, 0)),
                  pl.BlockSpec((BN, F), lambda i: (i, 0))],
        out_specs=pl.BlockSpec((BN, F), lambda i: (i, 0)),
        out_shape=jax.ShapeDtypeStruct((N, F), F32),
    )(x, q, out_parts, out_parts, inv)
    return out


# speedup vs baseline: 1.1305x; 1.1305x over previous
import functools

import jax
import jax.numpy as jnp
from jax import lax
from jax.experimental import pallas as pl
from jax.experimental.pallas import tpu as pltpu
from jax.experimental.pallas import tpu_sc as plsc

F32 = jnp.float32

N = 10000
E = 160000
F = 128
H = 4

NC = 2
NS = 16
NW = NC * NS
EPW = E // NW
CB = 40
NCH = EPW // CB
QW = H * F
ZCH = 64
NZC = -(-N // ZCH)
CB2 = 32
NCH2 = 156
XCH2 = 8


def _prep_kernel(wproj_ref, wout_ref, rs_ref, ts_ref, a2_ref, vt_ref):
    wout = wout_ref[...]
    cols_r = []
    cols_t = []
    for h in range(H):
        wh = wproj_ref[h]
        a2_ref[:, h * F:(h + 1) * F] = jnp.dot(
            wh, wout, preferred_element_type=F32)
        cols_r.append(jnp.dot(wh, rs_ref[h][:, None],
                              preferred_element_type=F32))
        cols_t.append(jnp.dot(wh, ts_ref[h][:, None],
                              preferred_element_type=F32))
    pad = jnp.zeros((F, F - 8), F32)
    vt_ref[...] = jnp.concatenate(cols_r + cols_t + [pad], axis=1)


def _proj_kernel(x_ref, a2_ref, vt_ref, q_ref, s_ref):
    xb = x_ref[...]
    q_ref[...] = jnp.dot(xb, a2_ref[...], preferred_element_type=F32)
    s_ref[...] = jnp.dot(xb, vt_ref[...], preferred_element_type=F32)


def _pass1_body(s_hbm, snd_hbm, rcv_hbm, len_hbm, ew_hbm, rsum_hbm,
                sidx0, sidx1, ridx0, ridx1, ssb0, ssb1, lenb0, lenb1,
                ewb, ewout, zb, accum, gsem0, gsem1, isem0, isem1):
    cid = lax.axis_index("c")
    sid = lax.axis_index("s")
    wid = sid * NC + cid

    zrow = jnp.zeros((16,), F32)

    def zero_wide(i, _):
        for j in range(8):
            zb[i, pl.ds(j * 16, 16)] = zrow
        return _

    def zero_ewb(i, _):
        for j in range(8):
            ewb[i, pl.ds(j * 16, 16)] = zrow
        return _

    lax.fori_loop(0, CB, zero_ewb, None)
    lax.fori_loop(0, ZCH, zero_wide, None)
    for c in range(NZC):
        sz = min(ZCH, N - c * ZCH)

        @pl.when(sid == c % NS)
        def _():
            pltpu.sync_copy(zb.at[pl.ds(0, sz)], accum.at[pl.ds(c * ZCH, sz)])
    plsc.subcore_barrier()

    base0 = wid * EPW
    mask = jnp.where(lax.broadcasted_iota(jnp.int32, (16,), 0) < 4,
                     jnp.float32(1.0), jnp.float32(0.0))
    slots = ((sidx0, ridx0, ssb0, lenb0, gsem0, isem0),
             (sidx1, ridx1, ssb1, lenb1, gsem1, isem1))

    def issue(c, slot):
        sidx, ridx, ssb, lenb, gsem, isem = slots[slot]
        base = base0 + c * CB
        d_s = pltpu.async_copy(snd_hbm.at[pl.ds(base, CB)], sidx, isem)
        pltpu.async_copy(rcv_hbm.at[pl.ds(base, CB)], ridx, gsem)
        pltpu.async_copy(len_hbm.at[pl.ds(base, CB)],
                         lenb.at[pl.ds(0, CB)], gsem)
        d_s.wait()
        pltpu.async_copy(s_hbm.at[sidx], ssb, gsem)

    def process(c, slot):
        sidx, ridx, ssb, lenb, gsem, isem = slots[slot]
        pltpu.make_async_copy(rcv_hbm.at[pl.ds(0, CB)], ridx, gsem).wait()
        pltpu.make_async_copy(len_hbm.at[pl.ds(0, CB)],
                              lenb.at[pl.ds(0, CB)], gsem).wait()
        pltpu.make_async_copy(s_hbm.at[pl.ds(0, CB)], ssb, gsem).wait()

        def edge(i, _):
            lv = lenb[pl.ds(i, 16)][0]
            row = jnp.exp(ssb[i, pl.ds(0, 16)] - lv * mask)
            ewb[i, pl.ds(0, 16)] = row
            ewout[i] = row
            return _

        lax.fori_loop(0, CB, edge, None)
        base = base0 + c * CB
        pltpu.sync_copy(ewout, ew_hbm.at[pl.ds(base, CB)])
        pltpu.sync_copy(ewb, accum.at[ridx], add=True)

        @pl.when(c + 2 < NCH)
        def _():
            issue(c + 2, slot)

    issue(0, 0)
    issue(1, 1)

    def pair(g, _):
        process(2 * g, 0)

        @pl.when(2 * g + 1 < NCH)
        def _():
            process(2 * g + 1, 1)
        return _

    lax.fori_loop(0, (NCH + 1) // 2, pair, None)
    plsc.subcore_barrier()
    for c in range(NZC):
        sz = min(ZCH, N - c * ZCH)

        @pl.when(sid == c % NS)
        def _():
            pltpu.sync_copy(accum.at[pl.ds(c * ZCH, sz)],
                            rsum_hbm.at[pl.ds(cid * N + c * ZCH, sz)])


def _combine_kernel(p0_ref, p1_ref, inv_ref):
    inv_ref[...] = 1.0 / (p0_ref[...] + p1_ref[...])


def _pass2_body(q_hbm, inv_hbm, ew_hbm, snd_hbm, rcv_hbm, out_hbm,
                sidx0, sidx1, ridx0, ridx1, qrows0, qrows1, ewb0, ewb1,
                invb, outb, accum, gsem0, gsem1, isem0, isem1):
    cid = lax.axis_index("c")
    sid = lax.axis_index("s")
    wid = sid * NC + cid
    nch = jnp.where(wid == NW - 1, NCH2 + XCH2, NCH2)

    zrow = jnp.zeros((16,), F32)

    def zero_row(i, _):
        for j in range(8):
            outb[i, pl.ds(j * 16, 16)] = zrow
        return _

    lax.fori_loop(0, CB2, zero_row, None)

    def zero_chunk(c, _):
        @pl.when(c % NS == sid)
        def _():
            pltpu.sync_copy(outb, accum.at[pl.ds(c * CB2, CB2)])
        return _

    lax.fori_loop(0, N // CB2, zero_chunk, None)
    if N % CB2:
        @pl.when(sid == (N // CB2) % NS)
        def _():
            pltpu.sync_copy(outb.at[pl.ds(0, N % CB2)],
                            accum.at[pl.ds(N - N % CB2, N % CB2)])
    plsc.subcore_barrier()

    base0 = wid * (NCH2 * CB2)
    slots = ((sidx0, ridx0, qrows0, ewb0, gsem0, isem0),
             (sidx1, ridx1, qrows1, ewb1, gsem1, isem1))

    def issue(c, slot):
        sidx, ridx, qrows, ewb, gsem, isem = slots[slot]
        base = base0 + c * CB2
        d_s = pltpu.async_copy(snd_hbm.at[pl.ds(base, CB2)], sidx, isem)
        d_r = pltpu.async_copy(rcv_hbm.at[pl.ds(base, CB2)], ridx, isem)
        pltpu.async_copy(ew_hbm.at[pl.ds(base, CB2)], ewb, gsem)
        d_s.wait()
        d_r.wait()
        pltpu.async_copy(q_hbm.at[sidx], qrows, gsem)

    def process(c, slot):
        sidx, ridx, qrows, ewb, gsem, isem = slots[slot]
        d_i = pltpu.async_copy(inv_hbm.at[ridx], invb, gsem)
        pltpu.make_async_copy(ew_hbm.at[pl.ds(0, CB2)], ewb, gsem).wait()
        pltpu.make_async_copy(q_hbm.at[pl.ds(0, CB2)], qrows, gsem).wait()
        d_i.wait()

        def edge(i, _):
            wv = ewb[i] * invb[i, pl.ds(0, 16)]
            w0 = wv[0] + wv[4]
            w1 = wv[1] + wv[5]
            w2 = wv[2] + wv[6]
            w3 = wv[3] + wv[7]
            for j in range(8):
                o = j * 16
                acc = (w0 * qrows[i, pl.ds(o, 16)]
                       + w1 * qrows[i, pl.ds(F + o, 16)]
                       + w2 * qrows[i, pl.ds(2 * F + o, 16)]
                       + w3 * qrows[i, pl.ds(3 * F + o, 16)])
                outb[i, pl.ds(o, 16)] = acc
            return _

        lax.fori_loop(0, CB2, edge, None)
        pltpu.sync_copy(outb, accum.at[ridx], add=True)

        @pl.when(c + 2 < nch)
        def _():
            issue(c + 2, slot)

    @pl.when(nch > 0)
    def _():
        issue(0, 0)

    @pl.when(nch > 1)
    def _():
        issue(1, 1)

    def pair(g, _):
        c = 2 * g

        @pl.when(c < nch)
        def _():
            process(c, 0)

        @pl.when(c + 1 < nch)
        def _():
            process(c + 1, 1)
        return _

    lax.fori_loop(0, (NCH2 + XCH2 + 1) // 2, pair, None)
    plsc.subcore_barrier()

    def dump_chunk(c, _):
        @pl.when(c % NS == sid)
        def _():
            pltpu.sync_copy(accum.at[pl.ds(c * CB2, CB2)],
                            out_hbm.at[pl.ds(cid * N + c * CB2, CB2)])
        return _

    lax.fori_loop(0, N // CB2, dump_chunk, None)
    if N % CB2:
        @pl.when(sid == (N // CB2) % NS)
        def _():
            pltpu.sync_copy(accum.at[pl.ds(N - N % CB2, N % CB2)],
                            out_hbm.at[pl.ds(cid * N + N - N % CB2,
                                             N % CB2)])


def _final_kernel(x_ref, q_ref, o0_ref, o1_ref, inv_ref, out_ref):
    ind = (inv_ref[...][:, :1] < jnp.inf).astype(F32)
    q = q_ref[...]
    sq = (q[:, 0 * F:1 * F] + q[:, 1 * F:2 * F]
          + q[:, 2 * F:3 * F] + q[:, 3 * F:4 * F])
    acc = o0_ref[...] + o1_ref[...] - 2.0 * ind * sq
    out_ref[...] = x_ref[...] + acc * (1.0 / H)


BN = 400


@jax.jit
def kernel(x, edge_index, edge_vec, edge_len, W_proj, W_out,
           radial_score, tangential_score, radial_distance_scale):
    del edge_vec
    snd = edge_index[0]
    rcv = edge_index[1]
    len2 = edge_len * radial_distance_scale

    a2, vt = pl.pallas_call(
        _prep_kernel,
        out_shape=(jax.ShapeDtypeStruct((F, QW), F32),
                   jax.ShapeDtypeStruct((F, F), F32)),
    )(W_proj, W_out, radial_score, tangential_score)

    nb = N // BN
    q, s = pl.pallas_call(
        _proj_kernel,
        grid=(nb,),
        in_specs=[pl.BlockSpec((BN, F), lambda i: (i, 0)),
                  pl.BlockSpec((F, QW), lambda i: (0, 0)),
                  pl.BlockSpec((F, F), lambda i: (0, 0))],
        out_specs=(pl.BlockSpec((BN, QW), lambda i: (i, 0)),
                   pl.BlockSpec((BN, F), lambda i: (i, 0))),
        out_shape=(jax.ShapeDtypeStruct((N, QW), F32),
                   jax.ShapeDtypeStruct((N, F), F32)),
    )(x, a2, vt)

    mesh = plsc.VectorSubcoreMesh(core_axis_name="c", subcore_axis_name="s")

    pass1 = functools.partial(
        pl.kernel,
        out_type=(jax.ShapeDtypeStruct((E, 16), F32),
                  jax.ShapeDtypeStruct((NC * N, F), F32)),
        mesh=mesh,
        scratch_types=[
            pltpu.VMEM((CB,), jnp.int32),
            pltpu.VMEM((CB,), jnp.int32),
            pltpu.VMEM((CB,), jnp.int32),
            pltpu.VMEM((CB,), jnp.int32),
            pltpu.VMEM((CB, F), F32),
            pltpu.VMEM((CB, F), F32),
            pltpu.VMEM((CB + 16,), F32),
            pltpu.VMEM((CB + 16,), F32),
            pltpu.VMEM((CB, F), F32),
            pltpu.VMEM((CB, 16), F32),
            pltpu.VMEM((ZCH, F), F32),
            pltpu.VMEM_SHARED((N, F), F32),
            pltpu.SemaphoreType.DMA,
            pltpu.SemaphoreType.DMA,
            pltpu.SemaphoreType.DMA,
            pltpu.SemaphoreType.DMA,
        ],
    )(_pass1_body)
    ew, rsum_parts = pass1(s, snd, rcv, len2)

    nb = N // BN
    inv = pl.pallas_call(
        _combine_kernel,
        grid=(nb,),
        in_specs=[pl.BlockSpec((BN, F), lambda i: (i, 0)),
                  pl.BlockSpec((BN, F), lambda i: (i + nb, 0))],
        out_specs=pl.BlockSpec((BN, F), lambda i: (i, 0)),
        out_shape=jax.ShapeDtypeStruct((N, F), F32),
    )(rsum_parts, rsum_parts)

    pass2 = functools.partial(
        pl.kernel,
        out_type=jax.ShapeDtypeStruct((NC * N, F), F32),
        mesh=mesh,
        scratch_types=[
            pltpu.VMEM((CB2,), jnp.int32),
            pltpu.VMEM((CB2,), jnp.int32),
            pltpu.VMEM((CB2,), jnp.int32),
            pltpu.VMEM((CB2,), jnp.int32),
            pltpu.VMEM((CB2, QW), F32),
            pltpu.VMEM((CB2, QW), F32),
            pltpu.VMEM((CB2, 16), F32),
            pltpu.VMEM((CB2, 16), F32),
            pltpu.VMEM((CB2, F), F32),
            pltpu.VMEM((CB2, F), F32),
            pltpu.VMEM_SHARED((N, F), F32),
            pltpu.SemaphoreType.DMA,
            pltpu.SemaphoreType.DMA,
            pltpu.SemaphoreType.DMA,
            pltpu.SemaphoreType.DMA,
        ],
    )(_pass2_body)
    out_parts = pass2(q, inv, ew, snd, rcv)

    out = pl.pallas_call(
        _final_kernel,
        grid=(nb,),
        in_specs=[pl.BlockSpec((BN, F), lambda i: (i, 0)),
                  pl.BlockSpec((BN, QW), lambda i: (i, 0)),
                  pl.BlockSpec((BN, F), lambda i: (i, 0)),
                  pl.BlockSpec((BN, F), lambda i: (i + nb, 0)),
                  pl.BlockSpec((BN, F), lambda i: (i, 0))],
        out_specs=pl.BlockSpec((BN, F), lambda i: (i, 0)),
        out_shape=jax.ShapeDtypeStruct((N, F), F32),
    )(x, q, out_parts, out_parts, inv)
    return out


# speedup vs baseline: 1.1712x; 1.0361x over previous
import functools

import jax
import jax.numpy as jnp
from jax import lax
from jax.experimental import pallas as pl
from jax.experimental.pallas import tpu as pltpu
from jax.experimental.pallas import tpu_sc as plsc

F32 = jnp.float32

N = 10000
E = 160000
F = 128
H = 4

NC = 2
NS = 16
NW = NC * NS
EPW = E // NW
CB = 40
NCH = EPW // CB
QW = H * F
ZCH = 64
NZC = -(-N // ZCH)
CB2 = 32
NCH2 = 156
XCH2 = 8


def _prep_kernel(wproj_ref, wout_ref, rs_ref, ts_ref, a2_ref, vt_ref):
    wout = wout_ref[...]
    cols_r = []
    cols_t = []
    for h in range(H):
        wh = wproj_ref[h]
        a2_ref[:, h * F:(h + 1) * F] = jnp.dot(
            wh, wout, preferred_element_type=F32)
        cols_r.append(jnp.dot(wh, rs_ref[h][:, None],
                              preferred_element_type=F32))
        cols_t.append(jnp.dot(wh, ts_ref[h][:, None],
                              preferred_element_type=F32))
    pad = jnp.zeros((F, F - 8), F32)
    vt_ref[...] = jnp.concatenate(cols_r + cols_t + [pad], axis=1)


def _proj_kernel(x_ref, a2_ref, vt_ref, q_ref, s_ref):
    xb = x_ref[...]
    q_ref[...] = jnp.dot(xb, a2_ref[...], preferred_element_type=F32)
    s_ref[...] = jnp.dot(xb, vt_ref[...], preferred_element_type=F32)


def _pass1_body(s_hbm, snd_hbm, rcv_hbm, len_hbm, ew_hbm, rsum_hbm,
                sidx0, sidx1, ridx0, ridx1, ssb0, ssb1, lenb0, lenb1,
                ewb, ewout, zb, accum, gsem0, gsem1, isem0, isem1):
    cid = lax.axis_index("c")
    sid = lax.axis_index("s")
    wid = sid * NC + cid

    zrow = jnp.zeros((16,), F32)

    def zero_wide(i, _):
        for j in range(8):
            zb[i, pl.ds(j * 16, 16)] = zrow
        return _

    def zero_ewb(i, _):
        for j in range(8):
            ewb[i, pl.ds(j * 16, 16)] = zrow
        return _

    lax.fori_loop(0, CB, zero_ewb, None)
    lax.fori_loop(0, ZCH, zero_wide, None)
    for c in range(NZC):
        sz = min(ZCH, N - c * ZCH)

        @pl.when(sid == c % NS)
        def _():
            pltpu.sync_copy(zb.at[pl.ds(0, sz)], accum.at[pl.ds(c * ZCH, sz)])
    plsc.subcore_barrier()

    base0 = wid * EPW
    mask = jnp.where(lax.broadcasted_iota(jnp.int32, (16,), 0) < 4,
                     jnp.float32(1.0), jnp.float32(0.0))
    slots = ((sidx0, ridx0, ssb0, lenb0, gsem0, isem0),
             (sidx1, ridx1, ssb1, lenb1, gsem1, isem1))

    def issue(c, slot):
        sidx, ridx, ssb, lenb, gsem, isem = slots[slot]
        base = base0 + c * CB
        d_s = pltpu.async_copy(snd_hbm.at[pl.ds(base, CB)], sidx, isem)
        pltpu.async_copy(rcv_hbm.at[pl.ds(base, CB)], ridx, gsem)
        pltpu.async_copy(len_hbm.at[pl.ds(base, CB)],
                         lenb.at[pl.ds(0, CB)], gsem)
        d_s.wait()
        pltpu.async_copy(s_hbm.at[sidx], ssb, gsem)

    def process(c, slot):
        sidx, ridx, ssb, lenb, gsem, isem = slots[slot]
        pltpu.make_async_copy(rcv_hbm.at[pl.ds(0, CB)], ridx, gsem).wait()
        pltpu.make_async_copy(len_hbm.at[pl.ds(0, CB)],
                              lenb.at[pl.ds(0, CB)], gsem).wait()
        pltpu.make_async_copy(s_hbm.at[pl.ds(0, CB)], ssb, gsem).wait()

        def edge(i, _):
            lv = lenb[pl.ds(i, 16)][0]
            row = jnp.exp(ssb[i, pl.ds(0, 16)] - lv * mask)
            ewb[i, pl.ds(0, 16)] = row
            ewout[i] = row
            return _

        lax.fori_loop(0, CB, edge, None)
        base = base0 + c * CB
        pltpu.sync_copy(ewout, ew_hbm.at[pl.ds(base, CB)])
        pltpu.sync_copy(ewb, accum.at[ridx], add=True)

        @pl.when(c + 2 < NCH)
        def _():
            issue(c + 2, slot)

    issue(0, 0)
    issue(1, 1)

    def pair(g, _):
        process(2 * g, 0)

        @pl.when(2 * g + 1 < NCH)
        def _():
            process(2 * g + 1, 1)
        return _

    lax.fori_loop(0, (NCH + 1) // 2, pair, None)
    plsc.subcore_barrier()
    for c in range(NZC):
        sz = min(ZCH, N - c * ZCH)

        @pl.when(sid == c % NS)
        def _():
            pltpu.sync_copy(accum.at[pl.ds(c * ZCH, sz)],
                            rsum_hbm.at[pl.ds(cid * N + c * ZCH, sz)])


def _combine_kernel(p0_ref, p1_ref, inv_ref):
    inv_ref[...] = 1.0 / (p0_ref[...] + p1_ref[...])


def _pass2_body(q_hbm, inv_hbm, ew_hbm, snd_hbm, rcv_hbm, out_hbm,
                sidx0, sidx1, ridx0, ridx1, ridxs0, ridxs1, qrows0, qrows1,
                ewb0, ewb1, invb, outb, accum, gsem0, gsem1, isem0, isem1,
                ssem):
    cid = lax.axis_index("c")
    sid = lax.axis_index("s")
    wid = sid * NC + cid
    nch = jnp.where(wid == NW - 1, NCH2 + XCH2, NCH2)

    zrow = jnp.zeros((16,), F32)

    def zero_row(i, _):
        for j in range(8):
            outb[i, pl.ds(j * 16, 16)] = zrow
        return _

    lax.fori_loop(0, CB2, zero_row, None)

    def zero_chunk(c, _):
        @pl.when(c % NS == sid)
        def _():
            pltpu.sync_copy(outb, accum.at[pl.ds(c * CB2, CB2)])
        return _

    lax.fori_loop(0, N // CB2, zero_chunk, None)
    if N % CB2:
        @pl.when(sid == (N // CB2) % NS)
        def _():
            pltpu.sync_copy(outb.at[pl.ds(0, N % CB2)],
                            accum.at[pl.ds(N - N % CB2, N % CB2)])
    plsc.subcore_barrier()

    base0 = wid * (NCH2 * CB2)
    slots = ((sidx0, ridx0, ridxs0, qrows0, ewb0, gsem0, isem0),
             (sidx1, ridx1, ridxs1, qrows1, ewb1, gsem1, isem1))

    def issue(c, slot):
        sidx, ridx, ridxs, qrows, ewb, gsem, isem = slots[slot]
        base = base0 + c * CB2
        d_s = pltpu.async_copy(snd_hbm.at[pl.ds(base, CB2)], sidx, isem)
        d_r = pltpu.async_copy(rcv_hbm.at[pl.ds(base, CB2)], ridx, isem)
        pltpu.async_copy(ew_hbm.at[pl.ds(base, CB2)], ewb, gsem)
        d_s.wait()
        d_r.wait()
        pltpu.async_copy(q_hbm.at[sidx], qrows, gsem)

    def process(c, slot):
        sidx, ridx, ridxs, qrows, ewb, gsem, isem = slots[slot]
        d_i = pltpu.async_copy(inv_hbm.at[ridx], invb, gsem)
        pltpu.make_async_copy(ew_hbm.at[pl.ds(0, CB2)], ewb, gsem).wait()
        pltpu.make_async_copy(q_hbm.at[pl.ds(0, CB2)], qrows, gsem).wait()
        d_i.wait()

        @pl.when(c >= 1)
        def _():
            pltpu.make_async_copy(inv_hbm.at[pl.ds(0, CB2)], outb,
                                  ssem).wait()

        def edge(i, _):
            wv = ewb[i] * invb[i, pl.ds(0, 16)]
            w0 = wv[0] + wv[4]
            w1 = wv[1] + wv[5]
            w2 = wv[2] + wv[6]
            w3 = wv[3] + wv[7]
            for j in range(8):
                o = j * 16
                acc = (w0 * qrows[i, pl.ds(o, 16)]
                       + w1 * qrows[i, pl.ds(F + o, 16)]
                       + w2 * qrows[i, pl.ds(2 * F + o, 16)]
                       + w3 * qrows[i, pl.ds(3 * F + o, 16)])
                outb[i, pl.ds(o, 16)] = acc
            return _

        lax.fori_loop(0, CB2, edge, None)
        ridxs[pl.ds(0, 16)] = ridx[pl.ds(0, 16)]
        ridxs[pl.ds(16, 16)] = ridx[pl.ds(16, 16)]
        pltpu.async_copy(outb, accum.at[ridxs], ssem, add=True)

        @pl.when(c + 2 < nch)
        def _():
            issue(c + 2, slot)

    @pl.when(nch > 0)
    def _():
        issue(0, 0)

    @pl.when(nch > 1)
    def _():
        issue(1, 1)

    def pair(g, _):
        c = 2 * g

        @pl.when(c < nch)
        def _():
            process(c, 0)

        @pl.when(c + 1 < nch)
        def _():
            process(c + 1, 1)
        return _

    lax.fori_loop(0, (NCH2 + XCH2 + 1) // 2, pair, None)
    pltpu.make_async_copy(inv_hbm.at[pl.ds(0, CB2)], outb, ssem).wait()
    plsc.subcore_barrier()

    def dump_chunk(c, _):
        @pl.when(c % NS == sid)
        def _():
            pltpu.sync_copy(accum.at[pl.ds(c * CB2, CB2)],
                            out_hbm.at[pl.ds(cid * N + c * CB2, CB2)])
        return _

    lax.fori_loop(0, N // CB2, dump_chunk, None)
    if N % CB2:
        @pl.when(sid == (N // CB2) % NS)
        def _():
            pltpu.sync_copy(accum.at[pl.ds(N - N % CB2, N % CB2)],
                            out_hbm.at[pl.ds(cid * N + N - N % CB2,
                                             N % CB2)])


def _final_kernel(x_ref, q_ref, o0_ref, o1_ref, inv_ref, out_ref):
    ind = (inv_ref[...][:, :1] < jnp.inf).astype(F32)
    q = q_ref[...]
    sq = (q[:, 0 * F:1 * F] + q[:, 1 * F:2 * F]
          + q[:, 2 * F:3 * F] + q[:, 3 * F:4 * F])
    acc = o0_ref[...] + o1_ref[...] - 2.0 * ind * sq
    out_ref[...] = x_ref[...] + acc * (1.0 / H)


BN = 400


@jax.jit
def kernel(x, edge_index, edge_vec, edge_len, W_proj, W_out,
           radial_score, tangential_score, radial_distance_scale):
    del edge_vec
    snd = edge_index[0]
    rcv = edge_index[1]
    len2 = edge_len * radial_distance_scale

    a2, vt = pl.pallas_call(
        _prep_kernel,
        out_shape=(jax.ShapeDtypeStruct((F, QW), F32),
                   jax.ShapeDtypeStruct((F, F), F32)),
    )(W_proj, W_out, radial_score, tangential_score)

    nb = N // BN
    q, s = pl.pallas_call(
        _proj_kernel,
        grid=(nb,),
        in_specs=[pl.BlockSpec((BN, F), lambda i: (i, 0)),
                  pl.BlockSpec((F, QW), lambda i: (0, 0)),
                  pl.BlockSpec((F, F), lambda i: (0, 0))],
        out_specs=(pl.BlockSpec((BN, QW), lambda i: (i, 0)),
                   pl.BlockSpec((BN, F), lambda i: (i, 0))),
        out_shape=(jax.ShapeDtypeStruct((N, QW), F32),
                   jax.ShapeDtypeStruct((N, F), F32)),
    )(x, a2, vt)

    mesh = plsc.VectorSubcoreMesh(core_axis_name="c", subcore_axis_name="s")

    pass1 = functools.partial(
        pl.kernel,
        out_type=(jax.ShapeDtypeStruct((E, 16), F32),
                  jax.ShapeDtypeStruct((NC * N, F), F32)),
        mesh=mesh,
        scratch_types=[
            pltpu.VMEM((CB,), jnp.int32),
            pltpu.VMEM((CB,), jnp.int32),
            pltpu.VMEM((CB,), jnp.int32),
            pltpu.VMEM((CB,), jnp.int32),
            pltpu.VMEM((CB, F), F32),
            pltpu.VMEM((CB, F), F32),
            pltpu.VMEM((CB + 16,), F32),
            pltpu.VMEM((CB + 16,), F32),
            pltpu.VMEM((CB, F), F32),
            pltpu.VMEM((CB, 16), F32),
            pltpu.VMEM((ZCH, F), F32),
            pltpu.VMEM_SHARED((N, F), F32),
            pltpu.SemaphoreType.DMA,
            pltpu.SemaphoreType.DMA,
            pltpu.SemaphoreType.DMA,
            pltpu.SemaphoreType.DMA,
        ],
    )(_pass1_body)
    ew, rsum_parts = pass1(s, snd, rcv, len2)

    nb = N // BN
    inv = pl.pallas_call(
        _combine_kernel,
        grid=(nb,),
        in_specs=[pl.BlockSpec((BN, F), lambda i: (i, 0)),
                  pl.BlockSpec((BN, F), lambda i: (i + nb, 0))],
        out_specs=pl.BlockSpec((BN, F), lambda i: (i, 0)),
        out_shape=jax.ShapeDtypeStruct((N, F), F32),
    )(rsum_parts, rsum_parts)

    pass2 = functools.partial(
        pl.kernel,
        out_type=jax.ShapeDtypeStruct((NC * N, F), F32),
        mesh=mesh,
        scratch_types=[
            pltpu.VMEM((CB2,), jnp.int32),
            pltpu.VMEM((CB2,), jnp.int32),
            pltpu.VMEM((CB2,), jnp.int32),
            pltpu.VMEM((CB2,), jnp.int32),
            pltpu.VMEM((CB2,), jnp.int32),
            pltpu.VMEM((CB2,), jnp.int32),
            pltpu.VMEM((CB2, QW), F32),
            pltpu.VMEM((CB2, QW), F32),
            pltpu.VMEM((CB2, 16), F32),
            pltpu.VMEM((CB2, 16), F32),
            pltpu.VMEM((CB2, F), F32),
            pltpu.VMEM((CB2, F), F32),
            pltpu.VMEM_SHARED((N, F), F32),
            pltpu.SemaphoreType.DMA,
            pltpu.SemaphoreType.DMA,
            pltpu.SemaphoreType.DMA,
            pltpu.SemaphoreType.DMA,
            pltpu.SemaphoreType.DMA,
        ],
    )(_pass2_body)
    out_parts = pass2(q, inv, ew, snd, rcv)

    out = pl.pallas_call(
        _final_kernel,
        grid=(nb,),
        in_specs=[pl.BlockSpec((BN, F), lambda i: (i, 0)),
                  pl.BlockSpec((BN, QW), lambda i: (i, 0)),
                  pl.BlockSpec((BN, F), lambda i: (i, 0)),
                  pl.BlockSpec((BN, F), lambda i: (i + nb, 0)),
                  pl.BlockSpec((BN, F), lambda i: (i, 0))],
        out_specs=pl.BlockSpec((BN, F), lambda i: (i, 0)),
        out_shape=jax.ShapeDtypeStruct((N, F), F32),
    )(x, q, out_parts, out_parts, inv)
    return out
